# baseline (speedup 1.0000x reference)
import jax
import jax.numpy as jnp
from jax.experimental import pallas as pl
from jax.experimental.pallas import tpu as pltpu

_CH = 512
_DEPTH = 6


def _body(x_hbm, w_hbm, o_hbm, xb, wb, ob, sem_w, sem_x, sem_o):
    B, S, D = x_hbm.shape
    PB = S // _CH
    N = B * PB

    def x_in(c, slot):
        b, r = divmod(c, PB)
        return pltpu.make_async_copy(
            x_hbm.at[b, pl.ds(r * _CH, _CH), :], xb.at[slot], sem_x.at[slot])

    def o_out(c, slot):
        b, r = divmod(c, PB)
        return pltpu.make_async_copy(
            ob.at[slot], o_hbm.at[b, pl.ds(r * _CH, _CH), :], sem_o.at[slot])

    pltpu.make_async_copy(w_hbm, wb, sem_w).start()
    for d in range(_DEPTH):
        x_in(d, d).start()
    pltpu.make_async_copy(w_hbm, wb, sem_w).wait()

    for c in range(N):
        slot = c % _DEPTH
        x_in(c, slot).wait()
        if c >= _DEPTH:
            o_out(c - _DEPTH, slot).wait()
        r = (c % PB) * _CH
        ob[slot] = xb[slot] + wb[pl.ds(r, _CH), :]
        o_out(c, slot).start()
        if c + _DEPTH < N:
            x_in(c + _DEPTH, slot).start()

    for c in range(N - _DEPTH, N):
        o_out(c, c % _DEPTH).wait()


def kernel(x, weight):
    B, S, D = x.shape
    w = weight[:S]
    return pl.pallas_call(
        _body,
        in_specs=[
            pl.BlockSpec(memory_space=pl.ANY),
            pl.BlockSpec(memory_space=pl.ANY),
        ],
        out_specs=pl.BlockSpec(memory_space=pl.ANY),
        out_shape=jax.ShapeDtypeStruct((B, S, D), x.dtype),
        scratch_shapes=[
            pltpu.VMEM((_DEPTH, _CH, D), x.dtype),
            pltpu.VMEM((S, D), x.dtype),
            pltpu.VMEM((_DEPTH, _CH, D), x.dtype),
            pltpu.SemaphoreType.DMA,
            pltpu.SemaphoreType.DMA((_DEPTH,)),
            pltpu.SemaphoreType.DMA((_DEPTH,)),
        ],
        compiler_params=pltpu.CompilerParams(vmem_limit_bytes=56 * 1024 * 1024),
    )(x, w)
